# baseline (speedup 1.0000x reference)
import jax
import jax.numpy as jnp
from jax.experimental import pallas as pl

B = 32768
H = 768
E = 64
G = 8
NUM_SUB = 4
BLK = 4096


def _router_kernel(x_ref, wg_ref, rm_ref, cm_ref, w_ref, ei_ref, si_ref):
    x = x_ref[...]
    wg = wg_ref[...]
    logits = jax.lax.dot_general(
        x, wg, (((1,), (1,)), ((), ())),
        preferred_element_type=jnp.float32)
    lt = logits.T

    row_mask = rm_ref[...]
    col_mask = cm_ref[...]

    gmax = jnp.max(lt, axis=0, keepdims=True)
    e = jnp.exp(lt - gmax)
    row_sum = jnp.dot(row_mask, e, preferred_element_type=jnp.float32)
    col_sum = jnp.dot(col_mask, e, preferred_element_type=jnp.float32)
    rs = row_sum / (row_sum * G)
    cs = col_sum / (col_sum * G)
    new_logits = rs * cs

    nmax = jnp.max(new_logits, axis=0, keepdims=True)
    f = jnp.exp(new_logits - nmax)
    p = f / jnp.sum(f, axis=0, keepdims=True)

    idx = jax.lax.broadcasted_iota(jnp.int32, (E, BLK), 0)
    m1 = jnp.max(p, axis=0, keepdims=True)
    i1 = jnp.min(jnp.where(p == m1, idx, E), axis=0, keepdims=True)
    p_masked = jnp.where(idx == i1, -1.0, p)
    m2 = jnp.max(p_masked, axis=0, keepdims=True)
    i2 = jnp.min(jnp.where(p_masked == m2, idx, E), axis=0, keepdims=True)

    denom = m1 + m2 + 1e-8
    w_ref[...] = jnp.concatenate([m1 / denom, m2 / denom], axis=0).T
    ei2 = jnp.concatenate([i1 // NUM_SUB, i2 // NUM_SUB], axis=0)
    si2 = jnp.concatenate([i1 % NUM_SUB, i2 % NUM_SUB], axis=0)
    ei_ref[...] = ei2.T
    si_ref[...] = si2.T


def kernel(x, W_gate):
    lane = jnp.arange(E)[:, None]
    lane_t = jnp.arange(E)[None, :]
    row_mask = (lane // G == lane_t // G).astype(jnp.float32)
    col_mask = (lane % G == lane_t % G).astype(jnp.float32)
    grid = (B // BLK,)
    w, ei, si = pl.pallas_call(
        _router_kernel,
        grid=grid,
        in_specs=[
            pl.BlockSpec((BLK, H), lambda i: (i, 0)),
            pl.BlockSpec((E, H), lambda i: (0, 0)),
            pl.BlockSpec((E, E), lambda i: (0, 0)),
            pl.BlockSpec((E, E), lambda i: (0, 0)),
        ],
        out_specs=[
            pl.BlockSpec((BLK, 2), lambda i: (i, 0)),
            pl.BlockSpec((BLK, 2), lambda i: (i, 0)),
            pl.BlockSpec((BLK, 2), lambda i: (i, 0)),
        ],
        out_shape=[
            jax.ShapeDtypeStruct((B, 2), jnp.float32),
            jax.ShapeDtypeStruct((B, 2), jnp.int32),
            jax.ShapeDtypeStruct((B, 2), jnp.int32),
        ],
    )(x, W_gate, row_mask, col_mask)
    loss = jnp.zeros((), dtype=jnp.float32)
    return (w, ei, si, loss)


# speedup vs baseline: 1.9243x; 1.9243x over previous
import jax
import jax.numpy as jnp
from jax.experimental import pallas as pl

B = 32768
H = 768
E = 64
G = 8
NUM_SUB = 4
BLK = 4096


def _router_kernel(x_ref, wg_ref, rm_ref, cm_ref, out_ref):
    x = x_ref[...]
    wg = wg_ref[...]
    logits = jax.lax.dot_general(
        x, wg, (((1,), (1,)), ((), ())),
        preferred_element_type=jnp.float32)
    lt = logits.T

    row_mask = rm_ref[...]
    col_mask = cm_ref[...]

    gmax = jnp.max(lt, axis=0, keepdims=True)
    e = jnp.exp(lt - gmax)
    row_sum = jnp.dot(row_mask, e, preferred_element_type=jnp.float32)
    col_sum = jnp.dot(col_mask, e, preferred_element_type=jnp.float32)
    rs = row_sum / (row_sum * G)
    cs = col_sum / (col_sum * G)
    new_logits = rs * cs

    nmax = jnp.max(new_logits, axis=0, keepdims=True)
    f = jnp.exp(new_logits - nmax)
    p = f / jnp.sum(f, axis=0, keepdims=True)

    idx = jax.lax.broadcasted_iota(jnp.int32, (E, BLK), 0)
    m1 = jnp.max(p, axis=0, keepdims=True)
    i1 = jnp.min(jnp.where(p == m1, idx, E), axis=0, keepdims=True)
    p_masked = jnp.where(idx == i1, -1.0, p)
    m2 = jnp.max(p_masked, axis=0, keepdims=True)
    i2 = jnp.min(jnp.where(p_masked == m2, idx, E), axis=0, keepdims=True)

    denom = m1 + m2 + 1e-8
    out_ref[...] = jnp.concatenate(
        [
            m1 / denom,
            m2 / denom,
            (i1 // NUM_SUB).astype(jnp.float32),
            (i2 // NUM_SUB).astype(jnp.float32),
            (i1 % NUM_SUB).astype(jnp.float32),
            (i2 % NUM_SUB).astype(jnp.float32),
            jnp.zeros((2, BLK), jnp.float32),
        ],
        axis=0,
    )


def kernel(x, W_gate):
    lane = jnp.arange(E)[:, None]
    lane_t = jnp.arange(E)[None, :]
    row_mask = (lane // G == lane_t // G).astype(jnp.float32)
    col_mask = (lane % G == lane_t % G).astype(jnp.float32)
    grid = (B // BLK,)
    packed = pl.pallas_call(
        _router_kernel,
        grid=grid,
        in_specs=[
            pl.BlockSpec((BLK, H), lambda i: (i, 0)),
            pl.BlockSpec((E, H), lambda i: (0, 0)),
            pl.BlockSpec((E, E), lambda i: (0, 0)),
            pl.BlockSpec((E, E), lambda i: (0, 0)),
        ],
        out_specs=pl.BlockSpec((8, BLK), lambda i: (0, i)),
        out_shape=jax.ShapeDtypeStruct((8, B), jnp.float32),
    )(x, W_gate, row_mask, col_mask)
    w = packed[0:2].T
    ei = packed[2:4].T.astype(jnp.int32)
    si = packed[4:6].T.astype(jnp.int32)
    loss = jnp.zeros((), dtype=jnp.float32)
    return (w, ei, si, loss)
